# baseline (speedup 1.0000x reference)
import functools

import jax
import jax.numpy as jnp
from jax.experimental import pallas as pl
from jax.experimental.pallas import tpu as pltpu
from jax.experimental.pallas import tpu_sc as plsc

NUM_CLASSES = 21
TOP_K = 200
NMS_THRESH = 0.45
CONF_THRESH = 0.01
V0 = 0.1
V1 = 0.2

B = 8
N = 20000
NP = 20480
R = B * NUM_CLASSES
RP = 176


def _k1_body(conf_ref, loc_ref, anc_ref, s_ref, boxes_ref, thr_ref, theta_ref):
    raw = conf_ref[...]
    s = jnp.where(raw > CONF_THRESH, raw, 0.0)
    s_ref[...] = s

    a_cx = anc_ref[0:1, :]
    a_cy = anc_ref[1:2, :]
    a_w = anc_ref[2:3, :]
    a_h = anc_ref[3:4, :]
    l0 = loc_ref[:, 0, :]
    l1 = loc_ref[:, 1, :]
    l2 = loc_ref[:, 2, :]
    l3 = loc_ref[:, 3, :]
    cx = a_cx + l0 * V0 * a_w
    cy = a_cy + l1 * V0 * a_h
    w = a_w * jnp.exp(l2 * V1)
    h = a_h * jnp.exp(l3 * V1)
    boxes_ref[:, 0, :] = cx - w * 0.5
    boxes_ref[:, 1, :] = cy - h * 0.5
    boxes_ref[:, 2, :] = cx + w * 0.5
    boxes_ref[:, 3, :] = cy + h * 0.5

    bits = jax.lax.bitcast_convert_type(s, jnp.int32)
    col = jax.lax.broadcasted_iota(jnp.int32, (RP, NP), 1)

    def vbody(_, lohi):
        lo, hi = lohi
        mid = jax.lax.shift_right_logical(lo + hi, 1)
        cnt = jnp.sum((bits >= mid).astype(jnp.int32), axis=1, keepdims=True)
        ge = cnt >= TOP_K
        return jnp.where(ge, mid, lo), jnp.where(ge, hi, mid)

    lo0 = jnp.zeros((RP, 1), jnp.int32)
    hi0 = jnp.full((RP, 1), 0x7F800000, jnp.int32)
    v200b, _ = jax.lax.fori_loop(0, 31, vbody, (lo0, hi0))

    t_need = TOP_K - jnp.sum((bits > v200b).astype(jnp.int32), axis=1,
                             keepdims=True)
    is_tie = bits == v200b

    def tbody(_, lohi):
        lo, hi = lohi
        mid = jax.lax.shift_right_logical(lo + hi, 1)
        g = jnp.sum((is_tie & (col < mid)).astype(jnp.int32), axis=1,
                    keepdims=True)
        ge = g >= t_need
        return jnp.where(ge, lo, mid), jnp.where(ge, mid, hi)

    lo1 = jnp.zeros((RP, 1), jnp.int32)
    hi1 = jnp.full((RP, 1), NP, jnp.int32)
    _, theta = jax.lax.fori_loop(0, 15, tbody, (lo1, hi1))
    theta = jnp.where(t_need <= 0, 0, theta)

    v200f = jax.lax.bitcast_convert_type(v200b, jnp.float32)
    thr_ref[...] = jnp.broadcast_to(v200f, (RP, 16))
    theta_ref[...] = jnp.broadcast_to(theta, (RP, 16))


def _k1_call(conf_t, loc_t, anchors_t):
    return pl.pallas_call(
        _k1_body,
        out_shape=(
            jax.ShapeDtypeStruct((RP, NP), jnp.float32),
            jax.ShapeDtypeStruct((B, 4, NP), jnp.float32),
            jax.ShapeDtypeStruct((RP, 16), jnp.float32),
            jax.ShapeDtypeStruct((RP, 16), jnp.int32),
        ),
    )(conf_t, loc_t, anchors_t)


def _k3_body(vals_ref, boxes_ref, outs_ref, outb_ref):
    x1 = boxes_ref[0]
    y1 = boxes_ref[1]
    x2 = boxes_ref[2]
    y2 = boxes_ref[3]
    vals = vals_ref[...]
    area = jnp.maximum(x2 - x1, 0.0) * jnp.maximum(y2 - y1, 0.0)
    ar = jax.lax.broadcasted_iota(jnp.int32, (TOP_K, R), 0)

    def body(i, keep):
        bx1 = boxes_ref[0, pl.ds(i, 1), :]
        by1 = boxes_ref[1, pl.ds(i, 1), :]
        bx2 = boxes_ref[2, pl.ds(i, 1), :]
        by2 = boxes_ref[3, pl.ds(i, 1), :]
        barea = (jnp.maximum(bx2 - bx1, 0.0) *
                 jnp.maximum(by2 - by1, 0.0))
        eq = jnp.where(ar == i, 1.0, 0.0)
        ki = jnp.sum(keep * eq, axis=0, keepdims=True) > 0.0
        xx1 = jnp.maximum(x1, bx1)
        yy1 = jnp.maximum(y1, by1)
        xx2 = jnp.minimum(x2, bx2)
        yy2 = jnp.minimum(y2, by2)
        inter = jnp.maximum(xx2 - xx1, 0.0) * jnp.maximum(yy2 - yy1, 0.0)
        union = area + barea - inter
        iou = inter / jnp.maximum(union, 1e-9)
        sup = (iou > NMS_THRESH) & (ar > i) & ki
        return jnp.where(sup, 0.0, keep)

    keep = jax.lax.fori_loop(0, TOP_K, body,
                             jnp.ones((TOP_K, R), dtype=jnp.float32))
    keep = (keep > 0.0) & (vals > 0.0)
    outs_ref[...] = jnp.where(keep, vals, 0.0)
    outb_ref[0] = jnp.where(keep, x1, 0.0)
    outb_ref[1] = jnp.where(keep, y1, 0.0)
    outb_ref[2] = jnp.where(keep, x2, 0.0)
    outb_ref[3] = jnp.where(keep, y2, 0.0)


def _k3_call(svals_t, sboxes_t):
    return pl.pallas_call(
        _k3_body,
        out_shape=(
            jax.ShapeDtypeStruct((TOP_K, R), jnp.float32),
            jax.ShapeDtypeStruct((4, TOP_K, R), jnp.float32),
        ),
    )(svals_t, sboxes_t)


KP = 224
NTILES = 32
TPI = 4
CLS_ITERS = (NUM_CLASSES + TPI - 1) // TPI


def _k2_body(s_hbm, thr_hbm, theta_hbm, boxes_hbm, svals_hbm, sboxes_hbm,
             row_v, thr_v, theta_v, cvals, cidx, svals_v, sidx_v,
             bx_v, by_v, bX_v, bY_v, g0, g1, g2, g3,
             q1v, q1i, q2v, q2i, q3v, q3i):
    nc = plsc.get_sparse_core_info().num_cores
    wid = jax.lax.axis_index("s") * nc + jax.lax.axis_index("c")
    img = wid // TPI
    coff = wid % TPI
    lane = jax.lax.broadcasted_iota(jnp.int32, (16,), 0)
    NQ = NP // 4
    NQB = NQ // 16

    pltpu.sync_copy(boxes_hbm.at[0, img], bx_v)
    pltpu.sync_copy(boxes_hbm.at[1, img], by_v)
    pltpu.sync_copy(boxes_hbm.at[2, img], bX_v)
    pltpu.sync_copy(boxes_hbm.at[3, img], bY_v)

    def do_row(r):
        pltpu.sync_copy(s_hbm.at[r], row_v)
        pltpu.sync_copy(thr_hbm.at[r], thr_v)
        pltpu.sync_copy(theta_hbm.at[r], theta_v)
        thrv = thr_v[...]
        thetav = theta_v[...]

        def cbody(i, offs):
            o0, o1, o2, o3 = offs

            def quarter(q, dstv, dsti, off):
                base = q * NQ + i * 16
                v = row_v[pl.ds(base, 16)]
                idxv = lane + base
                m = (v > thrv) | ((v == thrv) & (idxv < thetav))
                plsc.store_compressed(dstv.at[pl.ds(off, 16)], v, mask=m)
                plsc.store_compressed(dsti.at[pl.ds(off, 16)], idxv, mask=m)
                return off + jnp.sum(m.astype(jnp.int32))

            return (quarter(0, cvals, cidx, o0), quarter(1, q1v, q1i, o1),
                    quarter(2, q2v, q2i, o2), quarter(3, q3v, q3i, o3))

        c0, c1, c2, c3 = jax.lax.fori_loop(0, NQB, cbody, (0, 0, 0, 0))

        def mcopy(bufv, bufi, o, cnt):
            def mb(i, carry):
                @pl.when(i * 16 < cnt)
                def _():
                    cvals[pl.ds(o + i * 16, 16)] = bufv[pl.ds(i * 16, 16)]
                    cidx[pl.ds(o + i * 16, 16)] = bufi[pl.ds(i * 16, 16)]
                return carry

            jax.lax.fori_loop(0, (KP // 16) - 1, mb, 0)

        mcopy(q1v, q1i, c0, c1)
        mcopy(q2v, q2i, c0 + c1, c2)
        mcopy(q3v, q3i, c0 + c1 + c2, c3)

        cvals[pl.ds(200, 16)] = jnp.full((16,), -1.0, jnp.float32)
        cidx[pl.ds(200, 16)] = lane + 200
        cvals[pl.ds(208, 16)] = jnp.full((16,), -1.0, jnp.float32)
        cidx[pl.ds(208, 16)] = lane + 208

        for half in range(2):
            a0 = half * 7
            vas = [cvals[pl.ds((a0 + a) * 16, 16)] for a in range(7)]
            ias = [cidx[pl.ds((a0 + a) * 16, 16)] for a in range(7)]

            def jblk(jb, ranks):
                out = list(ranks)
                for l in range(16):
                    jv = jnp.zeros((16,), jnp.int32) + (jb * 16 + l)
                    bv = plsc.load_gather(cvals, [jv])
                    bi = plsc.load_gather(cidx, [jv])
                    for a in range(7):
                        gt = (bv > vas[a]) | ((bv == vas[a]) & (bi < ias[a]))
                        out[a] = out[a] + gt.astype(jnp.int32)
                return tuple(out)

            ranks = jax.lax.fori_loop(
                0, KP // 16, jblk, tuple(jnp.zeros((16,), jnp.int32)
                                         for _ in range(7)))
            for a in range(7):
                plsc.store_scatter(svals_v, [ranks[a]], vas[a])
                plsc.store_scatter(sidx_v, [ranks[a]], ias[a])

        def gbody(i, carry):
            idxv = sidx_v[pl.ds(i * 16, 16)]
            g0[pl.ds(i * 16, 16)] = plsc.load_gather(bx_v, [idxv])
            g1[pl.ds(i * 16, 16)] = plsc.load_gather(by_v, [idxv])
            g2[pl.ds(i * 16, 16)] = plsc.load_gather(bX_v, [idxv])
            g3[pl.ds(i * 16, 16)] = plsc.load_gather(bY_v, [idxv])
            return carry

        jax.lax.fori_loop(0, KP // 16, gbody, 0)

        pltpu.sync_copy(svals_v, svals_hbm.at[r])
        pltpu.sync_copy(g0, sboxes_hbm.at[0, r])
        pltpu.sync_copy(g1, sboxes_hbm.at[1, r])
        pltpu.sync_copy(g2, sboxes_hbm.at[2, r])
        pltpu.sync_copy(g3, sboxes_hbm.at[3, r])

    def rloop(k, carry):
        c = coff + TPI * k

        @pl.when(c < NUM_CLASSES)
        def _():
            do_row(img * NUM_CLASSES + c)

        return carry

    jax.lax.fori_loop(0, CLS_ITERS, rloop, 0)


def _k2_call(s, thr, theta, boxes4):
    mesh = plsc.VectorSubcoreMesh(core_axis_name="c", subcore_axis_name="s")
    f = pl.kernel(
        _k2_body,
        out_type=(
            jax.ShapeDtypeStruct((RP, KP), jnp.float32),
            jax.ShapeDtypeStruct((4, RP, KP), jnp.float32),
        ),
        mesh=mesh,
        compiler_params=pltpu.CompilerParams(needs_layout_passes=False),
        scratch_types=[
            pltpu.VMEM((NP,), jnp.float32),
            pltpu.VMEM((16,), jnp.float32),
            pltpu.VMEM((16,), jnp.int32),
            pltpu.VMEM((KP,), jnp.float32),
            pltpu.VMEM((KP,), jnp.int32),
            pltpu.VMEM((KP,), jnp.float32),
            pltpu.VMEM((KP,), jnp.int32),
            pltpu.VMEM((NP,), jnp.float32),
            pltpu.VMEM((NP,), jnp.float32),
            pltpu.VMEM((NP,), jnp.float32),
            pltpu.VMEM((NP,), jnp.float32),
            pltpu.VMEM((KP,), jnp.float32),
            pltpu.VMEM((KP,), jnp.float32),
            pltpu.VMEM((KP,), jnp.float32),
            pltpu.VMEM((KP,), jnp.float32),
            pltpu.VMEM((KP,), jnp.float32),
            pltpu.VMEM((KP,), jnp.int32),
            pltpu.VMEM((KP,), jnp.float32),
            pltpu.VMEM((KP,), jnp.int32),
            pltpu.VMEM((KP,), jnp.float32),
            pltpu.VMEM((KP,), jnp.int32),
        ],
    )
    return f(s, thr, theta, boxes4)


def _middle_jnp(s, thr, theta, boxes_flat):
    col = jnp.arange(NP, dtype=jnp.int32)[None, :]
    t = thr[:, :1]
    sel = (s > t) | ((s == t) & (col < theta[:, :1]))
    key = jnp.where(sel, s, -1.0)
    top_s, pos = jax.lax.top_k(key, TOP_K)
    vals = jnp.maximum(top_s, 0.0)
    b = jnp.clip(jnp.arange(RP, dtype=jnp.int32) // NUM_CLASSES, 0, B - 1)
    gidx = b[:, None] * NP + pos
    sboxes = boxes_flat[gidx]
    return vals, sboxes


def kernel(preds_loc, preds_conf, anchors):
    conf_t = jnp.transpose(preds_conf, (0, 2, 1)).reshape(R, N)
    conf_t = jnp.pad(conf_t, ((0, RP - R), (0, NP - N)))
    loc_t = jnp.pad(jnp.transpose(preds_loc, (0, 2, 1)),
                    ((0, 0), (0, 0), (0, NP - N)))
    anchors_t = jnp.pad(jnp.transpose(anchors, (1, 0)), ((0, 0), (0, NP - N)))

    s, boxes_t, thr, theta = _k1_call(conf_t, loc_t, anchors_t)

    boxes4 = jnp.transpose(boxes_t, (1, 0, 2))
    svals, sboxes = _k2_call(s, thr, theta, boxes4)
    svals_t = jnp.transpose(svals[:R, :TOP_K], (1, 0))
    sboxes_t = jnp.transpose(sboxes[:, :R, :TOP_K], (0, 2, 1))

    dets, detb = _k3_call(svals_t, sboxes_t)

    det_scores = jnp.transpose(dets, (1, 0)).reshape(B, NUM_CLASSES, TOP_K, 1)
    det_loc = jnp.transpose(detb, (2, 1, 0)).reshape(B, NUM_CLASSES, TOP_K, 4)
    return det_scores, det_loc


# speedup vs baseline: 2.3851x; 2.3851x over previous
import functools

import jax
import jax.numpy as jnp
from jax.experimental import pallas as pl
from jax.experimental.pallas import tpu as pltpu
from jax.experimental.pallas import tpu_sc as plsc

NUM_CLASSES = 21
TOP_K = 200
NMS_THRESH = 0.45
CONF_THRESH = 0.01
V0 = 0.1
V1 = 0.2

B = 8
N = 20000
NP = 20480
R = B * NUM_CLASSES
RP = 176


def _k1_body(conf_ref, loc_ref, anc_ref, s_ref, boxes_ref, thr_ref, theta_ref):
    raw = conf_ref[...]
    s = jnp.where(raw > CONF_THRESH, raw, 0.0)
    s_ref[...] = s

    a_cx = anc_ref[0:1, :]
    a_cy = anc_ref[1:2, :]
    a_w = anc_ref[2:3, :]
    a_h = anc_ref[3:4, :]
    l0 = loc_ref[:, 0, :]
    l1 = loc_ref[:, 1, :]
    l2 = loc_ref[:, 2, :]
    l3 = loc_ref[:, 3, :]
    cx = a_cx + l0 * V0 * a_w
    cy = a_cy + l1 * V0 * a_h
    w = a_w * jnp.exp(l2 * V1)
    h = a_h * jnp.exp(l3 * V1)
    boxes_ref[:, 0, :] = cx - w * 0.5
    boxes_ref[:, 1, :] = cy - h * 0.5
    boxes_ref[:, 2, :] = cx + w * 0.5
    boxes_ref[:, 3, :] = cy + h * 0.5

    bits = jax.lax.bitcast_convert_type(s, jnp.int32)
    col = jax.lax.broadcasted_iota(jnp.int32, (RP, NP), 1)

    def vbody(_, lohi):
        lo, hi = lohi
        mid = jax.lax.shift_right_logical(lo + hi, 1)
        cnt = jnp.sum((bits >= mid).astype(jnp.int32), axis=1, keepdims=True)
        ge = cnt >= TOP_K
        return jnp.where(ge, mid, lo), jnp.where(ge, hi, mid)

    lo0 = jnp.zeros((RP, 1), jnp.int32)
    hi0 = jnp.full((RP, 1), 0x7F800000, jnp.int32)
    v200b, _ = jax.lax.fori_loop(0, 31, vbody, (lo0, hi0))

    t_need = TOP_K - jnp.sum((bits > v200b).astype(jnp.int32), axis=1,
                             keepdims=True)
    is_tie = bits == v200b

    def tbody(_, lohi):
        lo, hi = lohi
        mid = jax.lax.shift_right_logical(lo + hi, 1)
        g = jnp.sum((is_tie & (col < mid)).astype(jnp.int32), axis=1,
                    keepdims=True)
        ge = g >= t_need
        return jnp.where(ge, lo, mid), jnp.where(ge, mid, hi)

    lo1 = jnp.zeros((RP, 1), jnp.int32)
    hi1 = jnp.full((RP, 1), NP, jnp.int32)
    _, theta = jax.lax.fori_loop(0, 15, tbody, (lo1, hi1))
    theta = jnp.where(t_need <= 0, 0, theta)

    v200f = jax.lax.bitcast_convert_type(v200b, jnp.float32)
    thr_ref[...] = jnp.broadcast_to(v200f, (RP, 16))
    theta_ref[...] = jnp.broadcast_to(theta, (RP, 16))


def _k1_call(conf_t, loc_t, anchors_t):
    return pl.pallas_call(
        _k1_body,
        out_shape=(
            jax.ShapeDtypeStruct((RP, NP), jnp.float32),
            jax.ShapeDtypeStruct((B, 4, NP), jnp.float32),
            jax.ShapeDtypeStruct((RP, 16), jnp.float32),
            jax.ShapeDtypeStruct((RP, 16), jnp.int32),
        ),
    )(conf_t, loc_t, anchors_t)


def _k3_body(vals_ref, boxes_ref, outs_ref, outb_ref):
    x1 = boxes_ref[0]
    y1 = boxes_ref[1]
    x2 = boxes_ref[2]
    y2 = boxes_ref[3]
    vals = vals_ref[...]
    area = jnp.maximum(x2 - x1, 0.0) * jnp.maximum(y2 - y1, 0.0)
    ar = jax.lax.broadcasted_iota(jnp.int32, (TOP_K, R), 0)

    def body(i, keep):
        bx1 = boxes_ref[0, pl.ds(i, 1), :]
        by1 = boxes_ref[1, pl.ds(i, 1), :]
        bx2 = boxes_ref[2, pl.ds(i, 1), :]
        by2 = boxes_ref[3, pl.ds(i, 1), :]
        barea = (jnp.maximum(bx2 - bx1, 0.0) *
                 jnp.maximum(by2 - by1, 0.0))
        eq = jnp.where(ar == i, 1.0, 0.0)
        ki = jnp.sum(keep * eq, axis=0, keepdims=True) > 0.0
        xx1 = jnp.maximum(x1, bx1)
        yy1 = jnp.maximum(y1, by1)
        xx2 = jnp.minimum(x2, bx2)
        yy2 = jnp.minimum(y2, by2)
        inter = jnp.maximum(xx2 - xx1, 0.0) * jnp.maximum(yy2 - yy1, 0.0)
        union = area + barea - inter
        iou = inter / jnp.maximum(union, 1e-9)
        sup = (iou > NMS_THRESH) & (ar > i) & ki
        return jnp.where(sup, 0.0, keep)

    keep = jax.lax.fori_loop(0, TOP_K, body,
                             jnp.ones((TOP_K, R), dtype=jnp.float32))
    keep = (keep > 0.0) & (vals > 0.0)
    outs_ref[...] = jnp.where(keep, vals, 0.0)
    outb_ref[0] = jnp.where(keep, x1, 0.0)
    outb_ref[1] = jnp.where(keep, y1, 0.0)
    outb_ref[2] = jnp.where(keep, x2, 0.0)
    outb_ref[3] = jnp.where(keep, y2, 0.0)


def _k3_call(svals_t, sboxes_t):
    return pl.pallas_call(
        _k3_body,
        out_shape=(
            jax.ShapeDtypeStruct((TOP_K, R), jnp.float32),
            jax.ShapeDtypeStruct((4, TOP_K, R), jnp.float32),
        ),
    )(svals_t, sboxes_t)


KP = 224
NTILES = 32
TPI = 4
CLS_ITERS = (NUM_CLASSES + TPI - 1) // TPI


def _k2_body(s_hbm, thr_hbm, theta_hbm, boxes_hbm, svals_hbm, sboxes_hbm,
             row_v, thr_v, theta_v, cvals, cidx, svals_v, sidx_v,
             bx_v, by_v, bX_v, bY_v, g0, g1, g2, g3,
             q1v, q1i, q2v, q2i, q3v, q3i):
    nc = plsc.get_sparse_core_info().num_cores
    wid = jax.lax.axis_index("s") * nc + jax.lax.axis_index("c")
    img = wid // TPI
    coff = wid % TPI
    lane = jax.lax.broadcasted_iota(jnp.int32, (16,), 0)
    NQ = NP // 4
    NQB = NQ // 16

    pltpu.sync_copy(boxes_hbm.at[0, img], bx_v)
    pltpu.sync_copy(boxes_hbm.at[1, img], by_v)
    pltpu.sync_copy(boxes_hbm.at[2, img], bX_v)
    pltpu.sync_copy(boxes_hbm.at[3, img], bY_v)

    def do_row(r):
        pltpu.sync_copy(s_hbm.at[r], row_v)
        pltpu.sync_copy(thr_hbm.at[r], thr_v)
        pltpu.sync_copy(theta_hbm.at[r], theta_v)
        thrv = thr_v[...]
        thetav = theta_v[...]

        def cbody(i, offs):
            o0, o1, o2, o3 = offs

            def quarter(q, dstv, dsti, off):
                base = q * NQ + i * 16
                v = row_v[pl.ds(base, 16)]
                idxv = lane + base
                m = (v > thrv) | ((v == thrv) & (idxv < thetav))
                plsc.store_compressed(dstv.at[pl.ds(off, 16)], v, mask=m)
                plsc.store_compressed(dsti.at[pl.ds(off, 16)], idxv, mask=m)
                return off + jnp.sum(m.astype(jnp.int32))

            return (quarter(0, cvals, cidx, o0), quarter(1, q1v, q1i, o1),
                    quarter(2, q2v, q2i, o2), quarter(3, q3v, q3i, o3))

        c0, c1, c2, c3 = jax.lax.fori_loop(0, NQB, cbody, (0, 0, 0, 0))

        def mcopy(bufv, bufi, o, cnt):
            def mb(i, carry):
                @pl.when(i * 16 < cnt)
                def _():
                    cvals[pl.ds(o + i * 16, 16)] = bufv[pl.ds(i * 16, 16)]
                    cidx[pl.ds(o + i * 16, 16)] = bufi[pl.ds(i * 16, 16)]
                return carry

            jax.lax.fori_loop(0, (KP // 16) - 1, mb, 0)

        mcopy(q1v, q1i, c0, c1)
        mcopy(q2v, q2i, c0 + c1, c2)
        mcopy(q3v, q3i, c0 + c1 + c2, c3)

        cvals[pl.ds(200, 16)] = jnp.full((16,), -1.0, jnp.float32)
        cidx[pl.ds(200, 16)] = lane + 200
        cvals[pl.ds(208, 16)] = jnp.full((16,), -1.0, jnp.float32)
        cidx[pl.ds(208, 16)] = lane + 208

        def sblk(a, carry):
            va = cvals[pl.ds(a * 16, 16)]
            ia = cidx[pl.ds(a * 16, 16)]

            def rbody(j, rank):
                jv = jnp.zeros((16,), jnp.int32) + j
                bv = plsc.load_gather(cvals, [jv])
                bi = plsc.load_gather(cidx, [jv])
                gt = (bv > va) | ((bv == va) & (bi < ia))
                return rank + gt.astype(jnp.int32)

            rank = jax.lax.fori_loop(0, KP, rbody,
                                     jnp.zeros((16,), jnp.int32))
            plsc.store_scatter(svals_v, [rank], va)
            plsc.store_scatter(sidx_v, [rank], ia)
            return carry

        jax.lax.fori_loop(0, KP // 16, sblk, 0)

        def gbody(i, carry):
            idxv = sidx_v[pl.ds(i * 16, 16)]
            g0[pl.ds(i * 16, 16)] = plsc.load_gather(bx_v, [idxv])
            g1[pl.ds(i * 16, 16)] = plsc.load_gather(by_v, [idxv])
            g2[pl.ds(i * 16, 16)] = plsc.load_gather(bX_v, [idxv])
            g3[pl.ds(i * 16, 16)] = plsc.load_gather(bY_v, [idxv])
            return carry

        jax.lax.fori_loop(0, KP // 16, gbody, 0)

        pltpu.sync_copy(svals_v, svals_hbm.at[r])
        pltpu.sync_copy(g0, sboxes_hbm.at[0, r])
        pltpu.sync_copy(g1, sboxes_hbm.at[1, r])
        pltpu.sync_copy(g2, sboxes_hbm.at[2, r])
        pltpu.sync_copy(g3, sboxes_hbm.at[3, r])

    def rloop(k, carry):
        c = coff + TPI * k

        @pl.when(c < NUM_CLASSES)
        def _():
            do_row(img * NUM_CLASSES + c)

        return carry

    jax.lax.fori_loop(0, CLS_ITERS, rloop, 0)


def _k2_call(s, thr, theta, boxes4):
    mesh = plsc.VectorSubcoreMesh(core_axis_name="c", subcore_axis_name="s")
    f = pl.kernel(
        _k2_body,
        out_type=(
            jax.ShapeDtypeStruct((RP, KP), jnp.float32),
            jax.ShapeDtypeStruct((4, RP, KP), jnp.float32),
        ),
        mesh=mesh,
        compiler_params=pltpu.CompilerParams(needs_layout_passes=False),
        scratch_types=[
            pltpu.VMEM((NP,), jnp.float32),
            pltpu.VMEM((16,), jnp.float32),
            pltpu.VMEM((16,), jnp.int32),
            pltpu.VMEM((KP,), jnp.float32),
            pltpu.VMEM((KP,), jnp.int32),
            pltpu.VMEM((KP,), jnp.float32),
            pltpu.VMEM((KP,), jnp.int32),
            pltpu.VMEM((NP,), jnp.float32),
            pltpu.VMEM((NP,), jnp.float32),
            pltpu.VMEM((NP,), jnp.float32),
            pltpu.VMEM((NP,), jnp.float32),
            pltpu.VMEM((KP,), jnp.float32),
            pltpu.VMEM((KP,), jnp.float32),
            pltpu.VMEM((KP,), jnp.float32),
            pltpu.VMEM((KP,), jnp.float32),
            pltpu.VMEM((KP,), jnp.float32),
            pltpu.VMEM((KP,), jnp.int32),
            pltpu.VMEM((KP,), jnp.float32),
            pltpu.VMEM((KP,), jnp.int32),
            pltpu.VMEM((KP,), jnp.float32),
            pltpu.VMEM((KP,), jnp.int32),
        ],
    )
    return f(s, thr, theta, boxes4)


def _middle_jnp(s, thr, theta, boxes_flat):
    col = jnp.arange(NP, dtype=jnp.int32)[None, :]
    t = thr[:, :1]
    sel = (s > t) | ((s == t) & (col < theta[:, :1]))
    key = jnp.where(sel, s, -1.0)
    top_s, pos = jax.lax.top_k(key, TOP_K)
    vals = jnp.maximum(top_s, 0.0)
    b = jnp.clip(jnp.arange(RP, dtype=jnp.int32) // NUM_CLASSES, 0, B - 1)
    gidx = b[:, None] * NP + pos
    sboxes = boxes_flat[gidx]
    return vals, sboxes


def kernel(preds_loc, preds_conf, anchors):
    conf_t = jnp.transpose(preds_conf, (0, 2, 1)).reshape(R, N)
    conf_t = jnp.pad(conf_t, ((0, RP - R), (0, NP - N)))
    loc_t = jnp.pad(jnp.transpose(preds_loc, (0, 2, 1)),
                    ((0, 0), (0, 0), (0, NP - N)))
    anchors_t = jnp.pad(jnp.transpose(anchors, (1, 0)), ((0, 0), (0, NP - N)))

    s, boxes_t, thr, theta = _k1_call(conf_t, loc_t, anchors_t)

    boxes4 = jnp.transpose(boxes_t, (1, 0, 2))
    svals, sboxes = _k2_call(s, thr, theta, boxes4)
    svals_t = jnp.transpose(svals[:R, :TOP_K], (1, 0))
    sboxes_t = jnp.transpose(sboxes[:, :R, :TOP_K], (0, 2, 1))

    dets, detb = _k3_call(svals_t, sboxes_t)

    det_scores = jnp.transpose(dets, (1, 0)).reshape(B, NUM_CLASSES, TOP_K, 1)
    det_loc = jnp.transpose(detb, (2, 1, 0)).reshape(B, NUM_CLASSES, TOP_K, 4)
    return det_scores, det_loc


# speedup vs baseline: 2.5725x; 1.0786x over previous
import functools

import jax
import jax.numpy as jnp
from jax.experimental import pallas as pl
from jax.experimental.pallas import tpu as pltpu
from jax.experimental.pallas import tpu_sc as plsc

NUM_CLASSES = 21
TOP_K = 200
NMS_THRESH = 0.45
CONF_THRESH = 0.01
V0 = 0.1
V1 = 0.2

B = 8
N = 20000
NP = 20480
R = B * NUM_CLASSES
RP = 176


def _k1_body(conf_ref, loc_ref, anc_ref, s_ref, boxes_ref, thr_ref, theta_ref):
    raw = conf_ref[...]
    s = jnp.where(raw > CONF_THRESH, raw, 0.0)
    s_ref[...] = s

    a_cx = anc_ref[0:1, :]
    a_cy = anc_ref[1:2, :]
    a_w = anc_ref[2:3, :]
    a_h = anc_ref[3:4, :]
    l0 = loc_ref[:, 0, :]
    l1 = loc_ref[:, 1, :]
    l2 = loc_ref[:, 2, :]
    l3 = loc_ref[:, 3, :]
    cx = a_cx + l0 * V0 * a_w
    cy = a_cy + l1 * V0 * a_h
    w = a_w * jnp.exp(l2 * V1)
    h = a_h * jnp.exp(l3 * V1)
    boxes_ref[:, 0, :] = cx - w * 0.5
    boxes_ref[:, 1, :] = cy - h * 0.5
    boxes_ref[:, 2, :] = cx + w * 0.5
    boxes_ref[:, 3, :] = cy + h * 0.5

    bits = jax.lax.bitcast_convert_type(s, jnp.int32)
    col = jax.lax.broadcasted_iota(jnp.int32, (RP, NP), 1)

    def vbody(_, lohi):
        lo, hi = lohi
        mid = jax.lax.shift_right_logical(lo + hi, 1)
        cnt = jnp.sum((bits >= mid).astype(jnp.int32), axis=1, keepdims=True)
        ge = cnt >= TOP_K
        return jnp.where(ge, mid, lo), jnp.where(ge, hi, mid)

    lo0 = jnp.zeros((RP, 1), jnp.int32)
    hi0 = jnp.full((RP, 1), 0x7F800000, jnp.int32)
    v200b, _ = jax.lax.fori_loop(0, 31, vbody, (lo0, hi0))

    t_need = TOP_K - jnp.sum((bits > v200b).astype(jnp.int32), axis=1,
                             keepdims=True)
    is_tie = bits == v200b

    def tbody(_, lohi):
        lo, hi = lohi
        mid = jax.lax.shift_right_logical(lo + hi, 1)
        g = jnp.sum((is_tie & (col < mid)).astype(jnp.int32), axis=1,
                    keepdims=True)
        ge = g >= t_need
        return jnp.where(ge, lo, mid), jnp.where(ge, mid, hi)

    lo1 = jnp.zeros((RP, 1), jnp.int32)
    hi1 = jnp.full((RP, 1), NP, jnp.int32)
    _, theta = jax.lax.fori_loop(0, 15, tbody, (lo1, hi1))
    theta = jnp.where(t_need <= 0, 0, theta)

    v200f = jax.lax.bitcast_convert_type(v200b, jnp.float32)
    thr_ref[...] = jnp.broadcast_to(v200f, (RP, 16))
    theta_ref[...] = jnp.broadcast_to(theta, (RP, 16))


def _k1_call(conf_t, loc_t, anchors_t):
    return pl.pallas_call(
        _k1_body,
        out_shape=(
            jax.ShapeDtypeStruct((RP, NP), jnp.float32),
            jax.ShapeDtypeStruct((B, 4, NP), jnp.float32),
            jax.ShapeDtypeStruct((RP, 16), jnp.float32),
            jax.ShapeDtypeStruct((RP, 16), jnp.int32),
        ),
    )(conf_t, loc_t, anchors_t)


def _k3_body(vals_ref, boxes_ref, outs_ref, outb_ref):
    x1 = boxes_ref[0]
    y1 = boxes_ref[1]
    x2 = boxes_ref[2]
    y2 = boxes_ref[3]
    vals = vals_ref[...]
    area = jnp.maximum(x2 - x1, 0.0) * jnp.maximum(y2 - y1, 0.0)
    ar = jax.lax.broadcasted_iota(jnp.int32, (TOP_K, R), 0)

    def body(i, keep):
        bx1 = boxes_ref[0, pl.ds(i, 1), :]
        by1 = boxes_ref[1, pl.ds(i, 1), :]
        bx2 = boxes_ref[2, pl.ds(i, 1), :]
        by2 = boxes_ref[3, pl.ds(i, 1), :]
        barea = (jnp.maximum(bx2 - bx1, 0.0) *
                 jnp.maximum(by2 - by1, 0.0))
        eq = jnp.where(ar == i, 1.0, 0.0)
        ki = jnp.sum(keep * eq, axis=0, keepdims=True) > 0.0
        xx1 = jnp.maximum(x1, bx1)
        yy1 = jnp.maximum(y1, by1)
        xx2 = jnp.minimum(x2, bx2)
        yy2 = jnp.minimum(y2, by2)
        inter = jnp.maximum(xx2 - xx1, 0.0) * jnp.maximum(yy2 - yy1, 0.0)
        union = area + barea - inter
        iou = inter / jnp.maximum(union, 1e-9)
        sup = (iou > NMS_THRESH) & (ar > i) & ki
        return jnp.where(sup, 0.0, keep)

    keep = jax.lax.fori_loop(0, TOP_K, body,
                             jnp.ones((TOP_K, R), dtype=jnp.float32))
    keep = (keep > 0.0) & (vals > 0.0)
    outs_ref[...] = jnp.where(keep, vals, 0.0)
    outb_ref[0] = jnp.where(keep, x1, 0.0)
    outb_ref[1] = jnp.where(keep, y1, 0.0)
    outb_ref[2] = jnp.where(keep, x2, 0.0)
    outb_ref[3] = jnp.where(keep, y2, 0.0)


def _k3_call(svals_t, sboxes_t):
    return pl.pallas_call(
        _k3_body,
        out_shape=(
            jax.ShapeDtypeStruct((TOP_K, R), jnp.float32),
            jax.ShapeDtypeStruct((4, TOP_K, R), jnp.float32),
        ),
    )(svals_t, sboxes_t)


KP = 224
NTILES = 32
TPI = 4
CLS_ITERS = (NUM_CLASSES + TPI - 1) // TPI


def _k2_body(s_hbm, thr_hbm, theta_hbm, boxes_hbm, svals_hbm, sboxes_hbm,
             row_v, thr_v, theta_v, cvals, cidx, svals_v, sidx_v,
             bx_v, by_v, bX_v, bY_v, g0, g1, g2, g3,
             q1v, q1i, q2v, q2i, q3v, q3i):
    nc = plsc.get_sparse_core_info().num_cores
    wid = jax.lax.axis_index("s") * nc + jax.lax.axis_index("c")
    img = wid // TPI
    coff = wid % TPI
    lane = jax.lax.broadcasted_iota(jnp.int32, (16,), 0)
    NQ = NP // 4
    NQB = NQ // 16

    pltpu.sync_copy(boxes_hbm.at[0, img], bx_v)
    pltpu.sync_copy(boxes_hbm.at[1, img], by_v)
    pltpu.sync_copy(boxes_hbm.at[2, img], bX_v)
    pltpu.sync_copy(boxes_hbm.at[3, img], bY_v)

    def do_row(r):
        pltpu.sync_copy(s_hbm.at[r], row_v)
        pltpu.sync_copy(thr_hbm.at[r], thr_v)
        pltpu.sync_copy(theta_hbm.at[r], theta_v)
        thrv = thr_v[...]
        thetav = theta_v[...]

        def cbody(i, offs):
            o0, o1, o2, o3 = offs

            def quarter(q, dstv, dsti, off):
                base = q * NQ + i * 16
                v = row_v[pl.ds(base, 16)]
                idxv = lane + base
                m = (v > thrv) | ((v == thrv) & (idxv < thetav))
                plsc.store_compressed(dstv.at[pl.ds(off, 16)], v, mask=m)
                plsc.store_compressed(dsti.at[pl.ds(off, 16)], idxv, mask=m)
                return off + jnp.sum(m.astype(jnp.int32))

            return (quarter(0, cvals, cidx, o0), quarter(1, q1v, q1i, o1),
                    quarter(2, q2v, q2i, o2), quarter(3, q3v, q3i, o3))

        c0, c1, c2, c3 = jax.lax.fori_loop(0, NQB, cbody, (0, 0, 0, 0))

        def mcopy(bufv, bufi, o, cnt):
            def mb(i, carry):
                @pl.when(i * 16 < cnt)
                def _():
                    cvals[pl.ds(o + i * 16, 16)] = bufv[pl.ds(i * 16, 16)]
                    cidx[pl.ds(o + i * 16, 16)] = bufi[pl.ds(i * 16, 16)]
                return carry

            jax.lax.fori_loop(0, (KP // 16) - 1, mb, 0)

        mcopy(q1v, q1i, c0, c1)
        mcopy(q2v, q2i, c0 + c1, c2)
        mcopy(q3v, q3i, c0 + c1 + c2, c3)

        cvals[pl.ds(200, 16)] = jnp.full((16,), -1.0, jnp.float32)
        cidx[pl.ds(200, 16)] = lane + 200
        cvals[pl.ds(208, 16)] = jnp.full((16,), -1.0, jnp.float32)
        cidx[pl.ds(208, 16)] = lane + 208

        def sblk(a, carry):
            va0 = cvals[pl.ds(a * 32, 16)]
            ia0 = cidx[pl.ds(a * 32, 16)]
            va1 = cvals[pl.ds(a * 32 + 16, 16)]
            ia1 = cidx[pl.ds(a * 32 + 16, 16)]

            def rbody(j, ranks):
                r0, r1 = ranks
                jv = jnp.zeros((16,), jnp.int32) + j
                bv = plsc.load_gather(cvals, [jv])
                bi = plsc.load_gather(cidx, [jv])
                g0_ = (bv > va0) | ((bv == va0) & (bi < ia0))
                g1_ = (bv > va1) | ((bv == va1) & (bi < ia1))
                return (r0 + g0_.astype(jnp.int32),
                        r1 + g1_.astype(jnp.int32))

            z = jnp.zeros((16,), jnp.int32)
            rank0, rank1 = jax.lax.fori_loop(0, KP, rbody, (z, z))
            plsc.store_scatter(svals_v, [rank0], va0)
            plsc.store_scatter(sidx_v, [rank0], ia0)
            plsc.store_scatter(svals_v, [rank1], va1)
            plsc.store_scatter(sidx_v, [rank1], ia1)
            return carry

        jax.lax.fori_loop(0, KP // 32, sblk, 0)

        def gbody(i, carry):
            idxv = sidx_v[pl.ds(i * 16, 16)]
            g0[pl.ds(i * 16, 16)] = plsc.load_gather(bx_v, [idxv])
            g1[pl.ds(i * 16, 16)] = plsc.load_gather(by_v, [idxv])
            g2[pl.ds(i * 16, 16)] = plsc.load_gather(bX_v, [idxv])
            g3[pl.ds(i * 16, 16)] = plsc.load_gather(bY_v, [idxv])
            return carry

        jax.lax.fori_loop(0, KP // 16, gbody, 0)

        pltpu.sync_copy(svals_v, svals_hbm.at[r])
        pltpu.sync_copy(g0, sboxes_hbm.at[0, r])
        pltpu.sync_copy(g1, sboxes_hbm.at[1, r])
        pltpu.sync_copy(g2, sboxes_hbm.at[2, r])
        pltpu.sync_copy(g3, sboxes_hbm.at[3, r])

    def rloop(k, carry):
        c = coff + TPI * k

        @pl.when(c < NUM_CLASSES)
        def _():
            do_row(img * NUM_CLASSES + c)

        return carry

    jax.lax.fori_loop(0, CLS_ITERS, rloop, 0)


def _k2_call(s, thr, theta, boxes4):
    mesh = plsc.VectorSubcoreMesh(core_axis_name="c", subcore_axis_name="s")
    f = pl.kernel(
        _k2_body,
        out_type=(
            jax.ShapeDtypeStruct((RP, KP), jnp.float32),
            jax.ShapeDtypeStruct((4, RP, KP), jnp.float32),
        ),
        mesh=mesh,
        compiler_params=pltpu.CompilerParams(needs_layout_passes=False),
        scratch_types=[
            pltpu.VMEM((NP,), jnp.float32),
            pltpu.VMEM((16,), jnp.float32),
            pltpu.VMEM((16,), jnp.int32),
            pltpu.VMEM((KP,), jnp.float32),
            pltpu.VMEM((KP,), jnp.int32),
            pltpu.VMEM((KP,), jnp.float32),
            pltpu.VMEM((KP,), jnp.int32),
            pltpu.VMEM((NP,), jnp.float32),
            pltpu.VMEM((NP,), jnp.float32),
            pltpu.VMEM((NP,), jnp.float32),
            pltpu.VMEM((NP,), jnp.float32),
            pltpu.VMEM((KP,), jnp.float32),
            pltpu.VMEM((KP,), jnp.float32),
            pltpu.VMEM((KP,), jnp.float32),
            pltpu.VMEM((KP,), jnp.float32),
            pltpu.VMEM((KP,), jnp.float32),
            pltpu.VMEM((KP,), jnp.int32),
            pltpu.VMEM((KP,), jnp.float32),
            pltpu.VMEM((KP,), jnp.int32),
            pltpu.VMEM((KP,), jnp.float32),
            pltpu.VMEM((KP,), jnp.int32),
        ],
    )
    return f(s, thr, theta, boxes4)


def _middle_jnp(s, thr, theta, boxes_flat):
    col = jnp.arange(NP, dtype=jnp.int32)[None, :]
    t = thr[:, :1]
    sel = (s > t) | ((s == t) & (col < theta[:, :1]))
    key = jnp.where(sel, s, -1.0)
    top_s, pos = jax.lax.top_k(key, TOP_K)
    vals = jnp.maximum(top_s, 0.0)
    b = jnp.clip(jnp.arange(RP, dtype=jnp.int32) // NUM_CLASSES, 0, B - 1)
    gidx = b[:, None] * NP + pos
    sboxes = boxes_flat[gidx]
    return vals, sboxes


def kernel(preds_loc, preds_conf, anchors):
    conf_t = jnp.transpose(preds_conf, (0, 2, 1)).reshape(R, N)
    conf_t = jnp.pad(conf_t, ((0, RP - R), (0, NP - N)))
    loc_t = jnp.pad(jnp.transpose(preds_loc, (0, 2, 1)),
                    ((0, 0), (0, 0), (0, NP - N)))
    anchors_t = jnp.pad(jnp.transpose(anchors, (1, 0)), ((0, 0), (0, NP - N)))

    s, boxes_t, thr, theta = _k1_call(conf_t, loc_t, anchors_t)

    boxes4 = jnp.transpose(boxes_t, (1, 0, 2))
    svals, sboxes = _k2_call(s, thr, theta, boxes4)
    svals_t = jnp.transpose(svals[:R, :TOP_K], (1, 0))
    sboxes_t = jnp.transpose(sboxes[:, :R, :TOP_K], (0, 2, 1))

    dets, detb = _k3_call(svals_t, sboxes_t)

    det_scores = jnp.transpose(dets, (1, 0)).reshape(B, NUM_CLASSES, TOP_K, 1)
    det_loc = jnp.transpose(detb, (2, 1, 0)).reshape(B, NUM_CLASSES, TOP_K, 4)
    return det_scores, det_loc


# speedup vs baseline: 2.6426x; 1.0272x over previous
import functools

import jax
import jax.numpy as jnp
from jax.experimental import pallas as pl
from jax.experimental.pallas import tpu as pltpu
from jax.experimental.pallas import tpu_sc as plsc

NUM_CLASSES = 21
TOP_K = 200
NMS_THRESH = 0.45
CONF_THRESH = 0.01
V0 = 0.1
V1 = 0.2

B = 8
N = 20000
NP = 20480
R = B * NUM_CLASSES
RP = 176


def _k1_body(conf_ref, loc_ref, anc_ref, s_ref, boxes_ref, thr_ref, theta_ref):
    raw = conf_ref[...]
    s = jnp.where(raw > CONF_THRESH, raw, 0.0)
    s_ref[...] = s

    a_cx = anc_ref[0:1, :]
    a_cy = anc_ref[1:2, :]
    a_w = anc_ref[2:3, :]
    a_h = anc_ref[3:4, :]
    l0 = loc_ref[:, 0, :]
    l1 = loc_ref[:, 1, :]
    l2 = loc_ref[:, 2, :]
    l3 = loc_ref[:, 3, :]
    cx = a_cx + l0 * V0 * a_w
    cy = a_cy + l1 * V0 * a_h
    w = a_w * jnp.exp(l2 * V1)
    h = a_h * jnp.exp(l3 * V1)
    boxes_ref[:, 0, :] = cx - w * 0.5
    boxes_ref[:, 1, :] = cy - h * 0.5
    boxes_ref[:, 2, :] = cx + w * 0.5
    boxes_ref[:, 3, :] = cy + h * 0.5

    bits = jax.lax.bitcast_convert_type(s, jnp.int32)
    col = jax.lax.broadcasted_iota(jnp.int32, (RP, NP), 1)

    def vbody(_, lohi):
        lo, hi = lohi
        mid = jax.lax.shift_right_logical(lo + hi, 1)
        cnt = jnp.sum((bits >= mid).astype(jnp.int32), axis=1, keepdims=True)
        ge = cnt >= TOP_K
        return jnp.where(ge, mid, lo), jnp.where(ge, hi, mid)

    lo0 = jnp.zeros((RP, 1), jnp.int32)
    hi0 = jnp.full((RP, 1), 0x7F800000, jnp.int32)
    v200b, _ = jax.lax.fori_loop(0, 31, vbody, (lo0, hi0))

    t_need = TOP_K - jnp.sum((bits > v200b).astype(jnp.int32), axis=1,
                             keepdims=True)
    is_tie = bits == v200b

    def tbody(_, lohi):
        lo, hi = lohi
        mid = jax.lax.shift_right_logical(lo + hi, 1)
        g = jnp.sum((is_tie & (col < mid)).astype(jnp.int32), axis=1,
                    keepdims=True)
        ge = g >= t_need
        return jnp.where(ge, lo, mid), jnp.where(ge, mid, hi)

    lo1 = jnp.zeros((RP, 1), jnp.int32)
    hi1 = jnp.full((RP, 1), NP, jnp.int32)
    _, theta = jax.lax.fori_loop(0, 15, tbody, (lo1, hi1))
    theta = jnp.where(t_need <= 0, 0, theta)

    v200f = jax.lax.bitcast_convert_type(v200b, jnp.float32)
    thr_ref[...] = jnp.broadcast_to(v200f, (RP, 16))
    theta_ref[...] = jnp.broadcast_to(theta, (RP, 16))


def _k1_call(conf_t, loc_t, anchors_t):
    return pl.pallas_call(
        _k1_body,
        out_shape=(
            jax.ShapeDtypeStruct((RP, NP), jnp.float32),
            jax.ShapeDtypeStruct((B, 4, NP), jnp.float32),
            jax.ShapeDtypeStruct((RP, 16), jnp.float32),
            jax.ShapeDtypeStruct((RP, 16), jnp.int32),
        ),
    )(conf_t, loc_t, anchors_t)


def _k3_body(vals_ref, boxes_ref, outs_ref, outb_ref):
    x1 = boxes_ref[0]
    y1 = boxes_ref[1]
    x2 = boxes_ref[2]
    y2 = boxes_ref[3]
    vals = vals_ref[...]
    area = jnp.maximum(x2 - x1, 0.0) * jnp.maximum(y2 - y1, 0.0)
    ar = jax.lax.broadcasted_iota(jnp.int32, (TOP_K, R), 0)

    def body(i, keep):
        bx1 = boxes_ref[0, pl.ds(i, 1), :]
        by1 = boxes_ref[1, pl.ds(i, 1), :]
        bx2 = boxes_ref[2, pl.ds(i, 1), :]
        by2 = boxes_ref[3, pl.ds(i, 1), :]
        barea = (jnp.maximum(bx2 - bx1, 0.0) *
                 jnp.maximum(by2 - by1, 0.0))
        eq = jnp.where(ar == i, 1.0, 0.0)
        ki = jnp.sum(keep * eq, axis=0, keepdims=True) > 0.0
        xx1 = jnp.maximum(x1, bx1)
        yy1 = jnp.maximum(y1, by1)
        xx2 = jnp.minimum(x2, bx2)
        yy2 = jnp.minimum(y2, by2)
        inter = jnp.maximum(xx2 - xx1, 0.0) * jnp.maximum(yy2 - yy1, 0.0)
        union = area + barea - inter
        iou = inter / jnp.maximum(union, 1e-9)
        sup = (iou > NMS_THRESH) & (ar > i) & ki
        return jnp.where(sup, 0.0, keep)

    keep = jax.lax.fori_loop(0, TOP_K, body,
                             jnp.ones((TOP_K, R), dtype=jnp.float32))
    keep = (keep > 0.0) & (vals > 0.0)
    outs_ref[...] = jnp.where(keep, vals, 0.0)
    outb_ref[0] = jnp.where(keep, x1, 0.0)
    outb_ref[1] = jnp.where(keep, y1, 0.0)
    outb_ref[2] = jnp.where(keep, x2, 0.0)
    outb_ref[3] = jnp.where(keep, y2, 0.0)


def _k3_call(svals_t, sboxes_t):
    return pl.pallas_call(
        _k3_body,
        out_shape=(
            jax.ShapeDtypeStruct((TOP_K, R), jnp.float32),
            jax.ShapeDtypeStruct((4, TOP_K, R), jnp.float32),
        ),
    )(svals_t, sboxes_t)


KP = 224
NTILES = 32
TPI = 4
CLS_ITERS = (NUM_CLASSES + TPI - 1) // TPI


def _k2_body(s_hbm, thr_hbm, theta_hbm, boxes_hbm, svals_hbm, sboxes_hbm,
             row_v, thr_v, theta_v, cvals, cidx, svals_v, sidx_v,
             bx_v, by_v, bX_v, bY_v, g0, g1, g2, g3,
             q1v, q1i, q2v, q2i, q3v, q3i):
    nc = plsc.get_sparse_core_info().num_cores
    wid = jax.lax.axis_index("s") * nc + jax.lax.axis_index("c")
    img = wid // TPI
    coff = wid % TPI
    lane = jax.lax.broadcasted_iota(jnp.int32, (16,), 0)
    NQ = NP // 4
    NQB = NQ // 16

    pltpu.sync_copy(boxes_hbm.at[0, img], bx_v)
    pltpu.sync_copy(boxes_hbm.at[1, img], by_v)
    pltpu.sync_copy(boxes_hbm.at[2, img], bX_v)
    pltpu.sync_copy(boxes_hbm.at[3, img], bY_v)

    def do_row(r):
        pltpu.sync_copy(s_hbm.at[r], row_v)
        pltpu.sync_copy(thr_hbm.at[r], thr_v)
        pltpu.sync_copy(theta_hbm.at[r], theta_v)
        thrv = thr_v[...]
        thetav = theta_v[...]

        def cbody(i, offs):
            o0, o1, o2, o3 = offs

            def quarter(q, dstv, dsti, off):
                base = q * NQ + i * 16
                v = row_v[pl.ds(base, 16)]
                idxv = lane + base
                m = (v > thrv) | ((v == thrv) & (idxv < thetav))
                plsc.store_compressed(dstv.at[pl.ds(off, 16)], v, mask=m)
                plsc.store_compressed(dsti.at[pl.ds(off, 16)], idxv, mask=m)
                return off + jnp.sum(m.astype(jnp.int32))

            return (quarter(0, cvals, cidx, o0), quarter(1, q1v, q1i, o1),
                    quarter(2, q2v, q2i, o2), quarter(3, q3v, q3i, o3))

        c0, c1, c2, c3 = jax.lax.fori_loop(0, NQB, cbody, (0, 0, 0, 0))

        def mcopy(bufv, bufi, o, cnt):
            def mb(i, carry):
                @pl.when(i * 16 < cnt)
                def _():
                    cvals[pl.ds(o + i * 16, 16)] = bufv[pl.ds(i * 16, 16)]
                    cidx[pl.ds(o + i * 16, 16)] = bufi[pl.ds(i * 16, 16)]
                return carry

            jax.lax.fori_loop(0, (KP // 16) - 1, mb, 0)

        mcopy(q1v, q1i, c0, c1)
        mcopy(q2v, q2i, c0 + c1, c2)
        mcopy(q3v, q3i, c0 + c1 + c2, c3)

        cvals[pl.ds(200, 16)] = jnp.full((16,), -1.0, jnp.float32)
        cidx[pl.ds(200, 16)] = lane + 200
        cvals[pl.ds(208, 16)] = jnp.full((16,), -1.0, jnp.float32)
        cidx[pl.ds(208, 16)] = lane + 208

        for blocks in ((0, 1, 2, 3), (4, 5, 6, 7), (8, 9, 10, 11), (12, 13)):
            vas = [cvals[pl.ds(a * 16, 16)] for a in blocks]
            ias = [cidx[pl.ds(a * 16, 16)] for a in blocks]

            def rbody(j, ranks, vas=vas, ias=ias):
                jv = jnp.zeros((16,), jnp.int32) + j
                bv = plsc.load_gather(cvals, [jv])
                bi = plsc.load_gather(cidx, [jv])
                out = []
                for va, ia, rk in zip(vas, ias, ranks):
                    gt = (bv > va) | ((bv == va) & (bi < ia))
                    out.append(rk + gt.astype(jnp.int32))
                return tuple(out)

            z = jnp.zeros((16,), jnp.int32)
            ranks = jax.lax.fori_loop(0, KP, rbody,
                                      tuple(z for _ in blocks))
            for va, ia, rk in zip(vas, ias, ranks):
                plsc.store_scatter(svals_v, [rk], va)
                plsc.store_scatter(sidx_v, [rk], ia)

        def gbody(i, carry):
            idxv = sidx_v[pl.ds(i * 16, 16)]
            g0[pl.ds(i * 16, 16)] = plsc.load_gather(bx_v, [idxv])
            g1[pl.ds(i * 16, 16)] = plsc.load_gather(by_v, [idxv])
            g2[pl.ds(i * 16, 16)] = plsc.load_gather(bX_v, [idxv])
            g3[pl.ds(i * 16, 16)] = plsc.load_gather(bY_v, [idxv])
            return carry

        jax.lax.fori_loop(0, KP // 16, gbody, 0)

        pltpu.sync_copy(svals_v, svals_hbm.at[r])
        pltpu.sync_copy(g0, sboxes_hbm.at[0, r])
        pltpu.sync_copy(g1, sboxes_hbm.at[1, r])
        pltpu.sync_copy(g2, sboxes_hbm.at[2, r])
        pltpu.sync_copy(g3, sboxes_hbm.at[3, r])

    def rloop(k, carry):
        c = coff + TPI * k

        @pl.when(c < NUM_CLASSES)
        def _():
            do_row(img * NUM_CLASSES + c)

        return carry

    jax.lax.fori_loop(0, CLS_ITERS, rloop, 0)


def _k2_call(s, thr, theta, boxes4):
    mesh = plsc.VectorSubcoreMesh(core_axis_name="c", subcore_axis_name="s")
    f = pl.kernel(
        _k2_body,
        out_type=(
            jax.ShapeDtypeStruct((RP, KP), jnp.float32),
            jax.ShapeDtypeStruct((4, RP, KP), jnp.float32),
        ),
        mesh=mesh,
        compiler_params=pltpu.CompilerParams(needs_layout_passes=False),
        scratch_types=[
            pltpu.VMEM((NP,), jnp.float32),
            pltpu.VMEM((16,), jnp.float32),
            pltpu.VMEM((16,), jnp.int32),
            pltpu.VMEM((KP,), jnp.float32),
            pltpu.VMEM((KP,), jnp.int32),
            pltpu.VMEM((KP,), jnp.float32),
            pltpu.VMEM((KP,), jnp.int32),
            pltpu.VMEM((NP,), jnp.float32),
            pltpu.VMEM((NP,), jnp.float32),
            pltpu.VMEM((NP,), jnp.float32),
            pltpu.VMEM((NP,), jnp.float32),
            pltpu.VMEM((KP,), jnp.float32),
            pltpu.VMEM((KP,), jnp.float32),
            pltpu.VMEM((KP,), jnp.float32),
            pltpu.VMEM((KP,), jnp.float32),
            pltpu.VMEM((KP,), jnp.float32),
            pltpu.VMEM((KP,), jnp.int32),
            pltpu.VMEM((KP,), jnp.float32),
            pltpu.VMEM((KP,), jnp.int32),
            pltpu.VMEM((KP,), jnp.float32),
            pltpu.VMEM((KP,), jnp.int32),
        ],
    )
    return f(s, thr, theta, boxes4)


def _middle_jnp(s, thr, theta, boxes_flat):
    col = jnp.arange(NP, dtype=jnp.int32)[None, :]
    t = thr[:, :1]
    sel = (s > t) | ((s == t) & (col < theta[:, :1]))
    key = jnp.where(sel, s, -1.0)
    top_s, pos = jax.lax.top_k(key, TOP_K)
    vals = jnp.maximum(top_s, 0.0)
    b = jnp.clip(jnp.arange(RP, dtype=jnp.int32) // NUM_CLASSES, 0, B - 1)
    gidx = b[:, None] * NP + pos
    sboxes = boxes_flat[gidx]
    return vals, sboxes


def kernel(preds_loc, preds_conf, anchors):
    conf_t = jnp.transpose(preds_conf, (0, 2, 1)).reshape(R, N)
    conf_t = jnp.pad(conf_t, ((0, RP - R), (0, NP - N)))
    loc_t = jnp.pad(jnp.transpose(preds_loc, (0, 2, 1)),
                    ((0, 0), (0, 0), (0, NP - N)))
    anchors_t = jnp.pad(jnp.transpose(anchors, (1, 0)), ((0, 0), (0, NP - N)))

    s, boxes_t, thr, theta = _k1_call(conf_t, loc_t, anchors_t)

    boxes4 = jnp.transpose(boxes_t, (1, 0, 2))
    svals, sboxes = _k2_call(s, thr, theta, boxes4)
    svals_t = jnp.transpose(svals[:R, :TOP_K], (1, 0))
    sboxes_t = jnp.transpose(sboxes[:, :R, :TOP_K], (0, 2, 1))

    dets, detb = _k3_call(svals_t, sboxes_t)

    det_scores = jnp.transpose(dets, (1, 0)).reshape(B, NUM_CLASSES, TOP_K, 1)
    det_loc = jnp.transpose(detb, (2, 1, 0)).reshape(B, NUM_CLASSES, TOP_K, 4)
    return det_scores, det_loc


# speedup vs baseline: 2.6434x; 1.0003x over previous
import jax
import jax.numpy as jnp
from jax.experimental import pallas as pl
from jax.experimental.pallas import tpu as pltpu
from jax.experimental.pallas import tpu_sc as plsc

NUM_CLASSES = 21
TOP_K = 200
NMS_THRESH = 0.45
CONF_THRESH = 0.01
V0 = 0.1
V1 = 0.2

B = 8
N = 20000
NP = 20480
R = B * NUM_CLASSES
RP = 176


def _k1_body(conf_ref, loc_ref, anc_ref, s_ref, boxes_ref, thr_ref, theta_ref):
    raw = conf_ref[...]
    s = jnp.where(raw > CONF_THRESH, raw, 0.0)
    s_ref[...] = s

    a_cx = anc_ref[0:1, :]
    a_cy = anc_ref[1:2, :]
    a_w = anc_ref[2:3, :]
    a_h = anc_ref[3:4, :]
    l0 = loc_ref[:, 0, :]
    l1 = loc_ref[:, 1, :]
    l2 = loc_ref[:, 2, :]
    l3 = loc_ref[:, 3, :]
    cx = a_cx + l0 * V0 * a_w
    cy = a_cy + l1 * V0 * a_h
    w = a_w * jnp.exp(l2 * V1)
    h = a_h * jnp.exp(l3 * V1)
    boxes_ref[:, 0, :] = cx - w * 0.5
    boxes_ref[:, 1, :] = cy - h * 0.5
    boxes_ref[:, 2, :] = cx + w * 0.5
    boxes_ref[:, 3, :] = cy + h * 0.5

    bits = jax.lax.bitcast_convert_type(s, jnp.int32)
    col = jax.lax.broadcasted_iota(jnp.int32, (RP, NP), 1)

    def vbody(_, lohi):
        lo, hi = lohi
        mid = jax.lax.shift_right_logical(lo + hi, 1)
        cnt = jnp.sum((bits >= mid).astype(jnp.int32), axis=1, keepdims=True)
        ge = cnt >= TOP_K
        return jnp.where(ge, mid, lo), jnp.where(ge, hi, mid)

    lo0 = jnp.zeros((RP, 1), jnp.int32)
    hi0 = jnp.full((RP, 1), 0x7F800000, jnp.int32)
    v200b, _ = jax.lax.fori_loop(0, 31, vbody, (lo0, hi0))

    t_need = TOP_K - jnp.sum((bits > v200b).astype(jnp.int32), axis=1,
                             keepdims=True)
    is_tie = bits == v200b

    def tbody(_, lohi):
        lo, hi = lohi
        mid = jax.lax.shift_right_logical(lo + hi, 1)
        g = jnp.sum((is_tie & (col < mid)).astype(jnp.int32), axis=1,
                    keepdims=True)
        ge = g >= t_need
        return jnp.where(ge, lo, mid), jnp.where(ge, mid, hi)

    lo1 = jnp.zeros((RP, 1), jnp.int32)
    hi1 = jnp.full((RP, 1), NP, jnp.int32)
    _, theta = jax.lax.fori_loop(0, 15, tbody, (lo1, hi1))
    theta = jnp.where(t_need <= 0, 0, theta)

    v200f = jax.lax.bitcast_convert_type(v200b, jnp.float32)
    thr_ref[...] = jnp.broadcast_to(v200f, (RP, 16))
    theta_ref[...] = jnp.broadcast_to(theta, (RP, 16))


def _k1_call(conf_t, loc_t, anchors_t):
    return pl.pallas_call(
        _k1_body,
        out_shape=(
            jax.ShapeDtypeStruct((RP, NP), jnp.float32),
            jax.ShapeDtypeStruct((B, 4, NP), jnp.float32),
            jax.ShapeDtypeStruct((RP, 16), jnp.float32),
            jax.ShapeDtypeStruct((RP, 16), jnp.int32),
        ),
    )(conf_t, loc_t, anchors_t)


def _k3_body(vals_ref, boxes_ref, outs_ref, outb_ref):
    x1 = boxes_ref[0]
    y1 = boxes_ref[1]
    x2 = boxes_ref[2]
    y2 = boxes_ref[3]
    vals = vals_ref[...]
    area = jnp.maximum(x2 - x1, 0.0) * jnp.maximum(y2 - y1, 0.0)
    ar = jax.lax.broadcasted_iota(jnp.int32, (TOP_K, R), 0)

    def body(i, keep):
        bx1 = boxes_ref[0, pl.ds(i, 1), :]
        by1 = boxes_ref[1, pl.ds(i, 1), :]
        bx2 = boxes_ref[2, pl.ds(i, 1), :]
        by2 = boxes_ref[3, pl.ds(i, 1), :]
        barea = (jnp.maximum(bx2 - bx1, 0.0) *
                 jnp.maximum(by2 - by1, 0.0))
        eq = jnp.where(ar == i, 1.0, 0.0)
        ki = jnp.sum(keep * eq, axis=0, keepdims=True) > 0.0
        xx1 = jnp.maximum(x1, bx1)
        yy1 = jnp.maximum(y1, by1)
        xx2 = jnp.minimum(x2, bx2)
        yy2 = jnp.minimum(y2, by2)
        inter = jnp.maximum(xx2 - xx1, 0.0) * jnp.maximum(yy2 - yy1, 0.0)
        union = area + barea - inter
        iou = inter / jnp.maximum(union, 1e-9)
        sup = (iou > NMS_THRESH) & (ar > i) & ki
        return jnp.where(sup, 0.0, keep)

    keep = jax.lax.fori_loop(0, TOP_K, body,
                             jnp.ones((TOP_K, R), dtype=jnp.float32))
    keep = (keep > 0.0) & (vals > 0.0)
    outs_ref[...] = jnp.where(keep, vals, 0.0)
    outb_ref[0] = jnp.where(keep, x1, 0.0)
    outb_ref[1] = jnp.where(keep, y1, 0.0)
    outb_ref[2] = jnp.where(keep, x2, 0.0)
    outb_ref[3] = jnp.where(keep, y2, 0.0)


def _k3_call(svals_t, sboxes_t):
    return pl.pallas_call(
        _k3_body,
        out_shape=(
            jax.ShapeDtypeStruct((TOP_K, R), jnp.float32),
            jax.ShapeDtypeStruct((4, TOP_K, R), jnp.float32),
        ),
    )(svals_t, sboxes_t)


KP = 224
NTILES = 32
TPI = 4
CLS_ITERS = (NUM_CLASSES + TPI - 1) // TPI


def _k2_body(s_hbm, thr_hbm, theta_hbm, boxes_hbm, svals_hbm, sboxes_hbm,
             row_v, thr_v, theta_v, cvals, cidx, svals_v, sidx_v,
             bx_v, by_v, bX_v, bY_v, g0, g1, g2, g3,
             q1v, q1i, q2v, q2i, q3v, q3i):
    nc = plsc.get_sparse_core_info().num_cores
    wid = jax.lax.axis_index("s") * nc + jax.lax.axis_index("c")
    img = wid // TPI
    coff = wid % TPI
    lane = jax.lax.broadcasted_iota(jnp.int32, (16,), 0)
    NQ = NP // 4
    NQB = NQ // 16

    pltpu.sync_copy(boxes_hbm.at[0, img], bx_v)
    pltpu.sync_copy(boxes_hbm.at[1, img], by_v)
    pltpu.sync_copy(boxes_hbm.at[2, img], bX_v)
    pltpu.sync_copy(boxes_hbm.at[3, img], bY_v)

    def do_row(r):
        pltpu.sync_copy(s_hbm.at[r], row_v)
        pltpu.sync_copy(thr_hbm.at[r], thr_v)
        pltpu.sync_copy(theta_hbm.at[r], theta_v)
        thrv = thr_v[...]
        thetav = theta_v[...]

        def cbody(i, offs):
            o0, o1, o2, o3 = offs

            def quarter(q, dstv, dsti, off):
                base = q * NQ + i * 16
                v = row_v[pl.ds(base, 16)]
                idxv = lane + base
                m = (v > thrv) | ((v == thrv) & (idxv < thetav))
                plsc.store_compressed(dstv.at[pl.ds(off, 16)], v, mask=m)
                plsc.store_compressed(dsti.at[pl.ds(off, 16)], idxv, mask=m)
                return off + jnp.sum(m.astype(jnp.int32))

            return (quarter(0, cvals, cidx, o0), quarter(1, q1v, q1i, o1),
                    quarter(2, q2v, q2i, o2), quarter(3, q3v, q3i, o3))

        c0, c1, c2, c3 = jax.lax.fori_loop(0, NQB, cbody, (0, 0, 0, 0))

        def mcopy(bufv, bufi, o, cnt):
            def mb(i, carry):
                @pl.when(i * 16 < cnt)
                def _():
                    cvals[pl.ds(o + i * 16, 16)] = bufv[pl.ds(i * 16, 16)]
                    cidx[pl.ds(o + i * 16, 16)] = bufi[pl.ds(i * 16, 16)]
                return carry

            jax.lax.fori_loop(0, (KP // 16) - 1, mb, 0)

        mcopy(q1v, q1i, c0, c1)
        mcopy(q2v, q2i, c0 + c1, c2)
        mcopy(q3v, q3i, c0 + c1 + c2, c3)

        cvals[pl.ds(200, 16)] = jnp.full((16,), -1.0, jnp.float32)
        cidx[pl.ds(200, 16)] = lane + 200
        cvals[pl.ds(208, 16)] = jnp.full((16,), -1.0, jnp.float32)
        cidx[pl.ds(208, 16)] = lane + 208

        for blocks in ((0, 1, 2, 3), (4, 5, 6, 7), (8, 9, 10, 11), (12, 13)):
            vas = [cvals[pl.ds(a * 16, 16)] for a in blocks]
            ias = [cidx[pl.ds(a * 16, 16)] for a in blocks]

            def rbody(j, ranks, vas=vas, ias=ias):
                jv = jnp.zeros((16,), jnp.int32) + j
                bv = plsc.load_gather(cvals, [jv])
                bi = plsc.load_gather(cidx, [jv])
                out = []
                for va, ia, rk in zip(vas, ias, ranks):
                    gt = (bv > va) | ((bv == va) & (bi < ia))
                    out.append(rk + gt.astype(jnp.int32))
                return tuple(out)

            z = jnp.zeros((16,), jnp.int32)
            ranks = jax.lax.fori_loop(0, KP, rbody,
                                      tuple(z for _ in blocks))
            for va, ia, rk in zip(vas, ias, ranks):
                plsc.store_scatter(svals_v, [rk], va)
                plsc.store_scatter(sidx_v, [rk], ia)

        def gbody(i, carry):
            idxv = sidx_v[pl.ds(i * 16, 16)]
            g0[pl.ds(i * 16, 16)] = plsc.load_gather(bx_v, [idxv])
            g1[pl.ds(i * 16, 16)] = plsc.load_gather(by_v, [idxv])
            g2[pl.ds(i * 16, 16)] = plsc.load_gather(bX_v, [idxv])
            g3[pl.ds(i * 16, 16)] = plsc.load_gather(bY_v, [idxv])
            return carry

        jax.lax.fori_loop(0, KP // 16, gbody, 0)

        pltpu.sync_copy(svals_v, svals_hbm.at[r])
        pltpu.sync_copy(g0, sboxes_hbm.at[0, r])
        pltpu.sync_copy(g1, sboxes_hbm.at[1, r])
        pltpu.sync_copy(g2, sboxes_hbm.at[2, r])
        pltpu.sync_copy(g3, sboxes_hbm.at[3, r])

    def rloop(k, carry):
        c = coff + TPI * k

        @pl.when(c < NUM_CLASSES)
        def _():
            do_row(img * NUM_CLASSES + c)

        return carry

    jax.lax.fori_loop(0, CLS_ITERS, rloop, 0)


def _k2_call(s, thr, theta, boxes4):
    mesh = plsc.VectorSubcoreMesh(core_axis_name="c", subcore_axis_name="s")
    f = pl.kernel(
        _k2_body,
        out_type=(
            jax.ShapeDtypeStruct((RP, KP), jnp.float32),
            jax.ShapeDtypeStruct((4, RP, KP), jnp.float32),
        ),
        mesh=mesh,
        compiler_params=pltpu.CompilerParams(needs_layout_passes=False),
        scratch_types=[
            pltpu.VMEM((NP,), jnp.float32),
            pltpu.VMEM((16,), jnp.float32),
            pltpu.VMEM((16,), jnp.int32),
            pltpu.VMEM((KP,), jnp.float32),
            pltpu.VMEM((KP,), jnp.int32),
            pltpu.VMEM((KP,), jnp.float32),
            pltpu.VMEM((KP,), jnp.int32),
            pltpu.VMEM((NP,), jnp.float32),
            pltpu.VMEM((NP,), jnp.float32),
            pltpu.VMEM((NP,), jnp.float32),
            pltpu.VMEM((NP,), jnp.float32),
            pltpu.VMEM((KP,), jnp.float32),
            pltpu.VMEM((KP,), jnp.float32),
            pltpu.VMEM((KP,), jnp.float32),
            pltpu.VMEM((KP,), jnp.float32),
            pltpu.VMEM((KP,), jnp.float32),
            pltpu.VMEM((KP,), jnp.int32),
            pltpu.VMEM((KP,), jnp.float32),
            pltpu.VMEM((KP,), jnp.int32),
            pltpu.VMEM((KP,), jnp.float32),
            pltpu.VMEM((KP,), jnp.int32),
        ],
    )
    return f(s, thr, theta, boxes4)


def kernel(preds_loc, preds_conf, anchors):
    conf_t = jnp.transpose(preds_conf, (0, 2, 1)).reshape(R, N)
    conf_t = jnp.pad(conf_t, ((0, RP - R), (0, NP - N)))
    loc_t = jnp.pad(jnp.transpose(preds_loc, (0, 2, 1)),
                    ((0, 0), (0, 0), (0, NP - N)))
    anchors_t = jnp.pad(jnp.transpose(anchors, (1, 0)), ((0, 0), (0, NP - N)))

    s, boxes_t, thr, theta = _k1_call(conf_t, loc_t, anchors_t)

    boxes4 = jnp.transpose(boxes_t, (1, 0, 2))
    svals, sboxes = _k2_call(s, thr, theta, boxes4)
    svals_t = jnp.transpose(svals[:R, :TOP_K], (1, 0))
    sboxes_t = jnp.transpose(sboxes[:, :R, :TOP_K], (0, 2, 1))

    dets, detb = _k3_call(svals_t, sboxes_t)

    det_scores = jnp.transpose(dets, (1, 0)).reshape(B, NUM_CLASSES, TOP_K, 1)
    det_loc = jnp.transpose(detb, (2, 1, 0)).reshape(B, NUM_CLASSES, TOP_K, 4)
    return det_scores, det_loc
